# baseline (speedup 1.0000x reference)
import jax
import jax.numpy as jnp
from jax import lax
from jax.experimental import pallas as pl

N = 8192
E = 8
D = 512
DH = D // 2
ROW = E + E * D
TB = 1024


def _body(p_ref, u_ref, e_ref, o_ref):
    p = p_ref[...]
    u = u_ref[...]
    eps = e_ref[...]
    logits = p[:, :E]
    uc = jnp.clip(u, 1e-6, 1.0 - 1e-6)
    g = -jnp.log(-jnp.log(uc))
    s = logits + g
    m = jnp.max(s, axis=1, keepdims=True)
    lane = lax.broadcasted_iota(jnp.int32, s.shape, 1)
    sel = jnp.min(jnp.where(s == m, lane, E), axis=1, keepdims=True)
    mean = jnp.zeros((TB, DH), jnp.float32)
    lstd = jnp.zeros((TB, DH), jnp.float32)
    for e in range(E):
        msk = sel == e
        mean = jnp.where(msk, p[:, E + e * D:E + e * D + DH], mean)
        lstd = jnp.where(msk, p[:, E + e * D + DH:E + (e + 1) * D], lstd)
    o_ref[...] = mean + eps * jnp.exp(lstd)


@jax.jit
def kernel(params, u, eps):
    return pl.pallas_call(
        _body,
        grid=(N // TB,),
        in_specs=[
            pl.BlockSpec((TB, ROW), lambda i: (i, 0)),
            pl.BlockSpec((TB, E), lambda i: (i, 0)),
            pl.BlockSpec((TB, DH), lambda i: (i, 0)),
        ],
        out_specs=pl.BlockSpec((TB, DH), lambda i: (i, 0)),
        out_shape=jax.ShapeDtypeStruct((N, DH), jnp.float32),
    )(params, u, eps)
